# baseline (speedup 1.0000x reference)
import jax
import jax.numpy as jnp
from jax import lax
from jax.experimental import pallas as pl
from jax.experimental.pallas import tpu as pltpu

E = 16
TOPK = 2
G = 8
KG = 4
ROUTE_SCALE = 2.5
D = 1024
F = 512
T = 1024

BLK = 128
NBLK = 32
_NEG = -1e30


def _gate_kernel(x_ref, gw_ref, xbf_ref, da_r_ref, db_r_ref, da_c_ref,
                 db_c_ref, wa_c_ref, wb_c_ref, meta_ref):
    x = x_ref[...]
    xbf_ref[...] = x.astype(jnp.bfloat16)
    scores = jax.nn.sigmoid(
        lax.dot_general(gw_ref[...], x, (((1,), (1,)), ((), ())),
                        preferred_element_type=jnp.float32))
    grows = [jnp.maximum(scores[2 * g:2 * g + 1, :], scores[2 * g + 1:2 * g + 2, :])
             for g in range(G)]
    gs = jnp.concatenate(grows, axis=0)
    iota_g = lax.broadcasted_iota(jnp.int32, (G, T), 0)
    keep = jnp.zeros((G, T), jnp.float32)
    work = gs
    for _ in range(KG):
        m = jnp.max(work, axis=0, keepdims=True)
        first = jnp.min(jnp.where(work == m, iota_g, G), axis=0, keepdims=True)
        sel = iota_g == first
        keep = keep + jnp.where(sel, 1.0, 0.0)
        work = jnp.where(sel, _NEG, work)
    keep_e = jnp.concatenate([keep[g:g + 1, :] for g in range(G) for _ in (0, 1)],
                             axis=0)
    masked = jnp.where(keep_e > 0.5, scores, _NEG)
    iota_e = lax.broadcasted_iota(jnp.int32, (E, T), 0)
    sels = []
    vals = []
    wsum = jnp.zeros((1, T), jnp.float32)
    work = masked
    for _ in range(TOPK):
        m = jnp.max(work, axis=0, keepdims=True)
        first = jnp.min(jnp.where(work == m, iota_e, E), axis=0, keepdims=True)
        sel = (iota_e == first)
        sels.append(jnp.where(sel, 1.0, 0.0))
        vals.append(m)
        wsum = wsum + m
        work = jnp.where(sel, _NEG, work)
    inv = ROUTE_SCALE / (wsum + 1e-20)
    wa = vals[0] * inv
    wb = vals[1] * inv

    mask = sels[0] + sels[1]
    csum = mask
    sh = 1
    while sh < T:
        shifted = jnp.concatenate(
            [jnp.zeros((E, sh), jnp.float32), csum[:, :T - sh]], axis=1)
        csum = csum + shifted
        sh *= 2
    rank = csum - mask
    counts = csum[:, T - 1:T]
    padded = jnp.floor((counts + (BLK - 1)) * (1.0 / BLK)) * BLK
    ii = lax.broadcasted_iota(jnp.int32, (E, E), 0)
    jj = lax.broadcasted_iota(jnp.int32, (E, E), 1)
    lt = jnp.where(ii > jj, 1.0, 0.0)
    off = lax.dot_general(lt, padded, (((1,), (0,)), ((), ())),
                          preferred_element_type=jnp.float32)
    dest = off + rank
    da = jnp.sum(sels[0] * dest, axis=0, keepdims=True)
    db = jnp.sum(sels[1] * dest, axis=0, keepdims=True)
    da_i = da.astype(jnp.int32)
    db_i = db.astype(jnp.int32)
    da_r_ref[...] = da_i
    db_r_ref[...] = db_i
    da_c_ref[...] = jnp.transpose(da_i)
    db_c_ref[...] = jnp.transpose(db_i)
    wa_c_ref[...] = jnp.transpose(wa)
    wb_c_ref[...] = jnp.transpose(wb)

    bio = (lax.broadcasted_iota(jnp.int32, (E, NBLK), 1) * BLK).astype(jnp.float32)
    blk_e = (jnp.sum(jnp.where(off <= bio, 1.0, 0.0), axis=0, keepdims=True)
             - 1.0)
    nblk = jnp.sum(padded, axis=0, keepdims=True) * (1.0 / BLK)
    meta_ref[...] = jnp.concatenate(
        [blk_e, nblk, jnp.zeros((1, NBLK - 1), jnp.float32)],
        axis=1).astype(jnp.int32)


def _mlp(xbf, w1, w3, w2):
    h1 = lax.dot_general(xbf, w1.astype(jnp.bfloat16), (((1,), (1,)), ((), ())),
                         preferred_element_type=jnp.float32)
    h3 = lax.dot_general(xbf, w3.astype(jnp.bfloat16), (((1,), (1,)), ((), ())),
                         preferred_element_type=jnp.float32)
    act = (h1 * jax.nn.sigmoid(h1) * h3).astype(jnp.bfloat16)
    return lax.dot_general(act, w2.astype(jnp.bfloat16), (((1,), (1,)), ((), ())),
                           preferred_element_type=jnp.float32)


def _moe_kernel(meta_ref, xbf_ref, w1_ref, w3_ref, w2_ref, ws1_ref, ws3_ref,
                ws2_ref, da_r_ref, db_r_ref, da_c_ref, db_c_ref, wa_c_ref,
                wb_c_ref, y_ref):
    s = pl.program_id(0)

    @pl.when(s == 0)
    def _():
        y_ref[...] = _mlp(xbf_ref[...], ws1_ref[...], ws3_ref[...],
                          ws2_ref[...])

    @pl.when(jnp.logical_and(s > 0, s - 1 < meta_ref[NBLK]))
    def _():
        base = (s - 1) * BLK
        rio = lax.broadcasted_iota(jnp.int32, (BLK, T), 0) + base
        pa = rio == da_r_ref[...]
        pb = rio == db_r_ref[...]
        p = (jnp.where(pa, 1.0, 0.0) + jnp.where(pb, 1.0, 0.0))
        xs = lax.dot_general(p.astype(jnp.bfloat16), xbf_ref[...],
                             (((1,), (0,)), ((), ())),
                             preferred_element_type=jnp.float32)
        out = _mlp(xs.astype(jnp.bfloat16), w1_ref[0], w3_ref[0], w2_ref[0])
        cval = (lax.dot_general(jnp.where(pa, 1.0, 0.0), wa_c_ref[...],
                                (((1,), (0,)), ((), ())),
                                preferred_element_type=jnp.float32)
                + lax.dot_general(jnp.where(pb, 1.0, 0.0), wb_c_ref[...],
                                  (((1,), (0,)), ((), ())),
                                  preferred_element_type=jnp.float32))
        out_s = out * cval
        cio = lax.broadcasted_iota(jnp.int32, (T, BLK), 1) + base
        qa = cio == da_c_ref[...]
        qb = cio == db_c_ref[...]
        q = jnp.where(qa, 1.0, 0.0) + jnp.where(qb, 1.0, 0.0)
        y_ref[...] = y_ref[...] + lax.dot_general(
            q, out_s, (((1,), (0,)), ((), ())),
            preferred_element_type=jnp.float32)


@jax.jit
def kernel(x, gate_w, W1, W2, W3, Ws1, Ws2, Ws3):
    xbf, da_r, db_r, da_c, db_c, wa_c, wb_c, meta = pl.pallas_call(
        _gate_kernel,
        out_shape=(
            jax.ShapeDtypeStruct((T, D), jnp.bfloat16),
            jax.ShapeDtypeStruct((1, T), jnp.int32),
            jax.ShapeDtypeStruct((1, T), jnp.int32),
            jax.ShapeDtypeStruct((T, 1), jnp.int32),
            jax.ShapeDtypeStruct((T, 1), jnp.int32),
            jax.ShapeDtypeStruct((T, 1), jnp.float32),
            jax.ShapeDtypeStruct((T, 1), jnp.float32),
            jax.ShapeDtypeStruct((1, 2 * NBLK), jnp.int32),
        ),
    )(x, gate_w)

    grid_spec = pltpu.PrefetchScalarGridSpec(
        num_scalar_prefetch=1,
        grid=(NBLK + 1,),
        in_specs=[
            pl.BlockSpec((T, D), lambda s, m: (0, 0)),
            pl.BlockSpec((1, F, D),
                         lambda s, m: (m[jnp.where(s > 0, s - 1, 0)], 0, 0)),
            pl.BlockSpec((1, F, D),
                         lambda s, m: (m[jnp.where(s > 0, s - 1, 0)], 0, 0)),
            pl.BlockSpec((1, D, F),
                         lambda s, m: (m[jnp.where(s > 0, s - 1, 0)], 0, 0)),
            pl.BlockSpec((F, D), lambda s, m: (0, 0)),
            pl.BlockSpec((F, D), lambda s, m: (0, 0)),
            pl.BlockSpec((D, F), lambda s, m: (0, 0)),
            pl.BlockSpec((1, T), lambda s, m: (0, 0)),
            pl.BlockSpec((1, T), lambda s, m: (0, 0)),
            pl.BlockSpec((T, 1), lambda s, m: (0, 0)),
            pl.BlockSpec((T, 1), lambda s, m: (0, 0)),
            pl.BlockSpec((T, 1), lambda s, m: (0, 0)),
            pl.BlockSpec((T, 1), lambda s, m: (0, 0)),
        ],
        out_specs=pl.BlockSpec((T, D), lambda s, m: (0, 0)),
    )
    y = pl.pallas_call(
        _moe_kernel,
        grid_spec=grid_spec,
        out_shape=jax.ShapeDtypeStruct((T, D), jnp.float32),
        compiler_params=pltpu.CompilerParams(
            dimension_semantics=("arbitrary",)),
    )(meta.reshape(2 * NBLK), xbf, W1, W3, W2, Ws1, Ws3, Ws2,
      da_r, db_r, da_c, db_c, wa_c, wb_c)
    return y


# speedup vs baseline: 1.0418x; 1.0418x over previous
import jax
import jax.numpy as jnp
from jax import lax
from jax.experimental import pallas as pl
from jax.experimental.pallas import tpu as pltpu

E = 16
TOPK = 2
G = 8
KG = 4
ROUTE_SCALE = 2.5
D = 1024
F = 512
T = 1024

BLK = 128
NBLK = 32
_NEG = -1e30


def _gate_kernel(x_ref, gw_ref, xbf_ref, da_r_ref, db_r_ref, da_c_ref,
                 db_c_ref, wa_c_ref, wb_c_ref, meta_ref):
    x = x_ref[...]
    xbf_ref[...] = x.astype(jnp.bfloat16)
    scores = jax.nn.sigmoid(
        lax.dot_general(gw_ref[...], x, (((1,), (1,)), ((), ())),
                        preferred_element_type=jnp.float32))
    grows = [jnp.maximum(scores[2 * g:2 * g + 1, :], scores[2 * g + 1:2 * g + 2, :])
             for g in range(G)]
    gs = jnp.concatenate(grows, axis=0)
    iota_g = lax.broadcasted_iota(jnp.int32, (G, T), 0)
    keep = jnp.zeros((G, T), jnp.float32)
    work = gs
    for _ in range(KG):
        m = jnp.max(work, axis=0, keepdims=True)
        first = jnp.min(jnp.where(work == m, iota_g, G), axis=0, keepdims=True)
        sel = iota_g == first
        keep = keep + jnp.where(sel, 1.0, 0.0)
        work = jnp.where(sel, _NEG, work)
    keep_e = jnp.concatenate([keep[g:g + 1, :] for g in range(G) for _ in (0, 1)],
                             axis=0)
    masked = jnp.where(keep_e > 0.5, scores, _NEG)
    iota_e = lax.broadcasted_iota(jnp.int32, (E, T), 0)
    sels = []
    vals = []
    wsum = jnp.zeros((1, T), jnp.float32)
    work = masked
    for _ in range(TOPK):
        m = jnp.max(work, axis=0, keepdims=True)
        first = jnp.min(jnp.where(work == m, iota_e, E), axis=0, keepdims=True)
        sel = (iota_e == first)
        sels.append(jnp.where(sel, 1.0, 0.0))
        vals.append(m)
        wsum = wsum + m
        work = jnp.where(sel, _NEG, work)
    inv = ROUTE_SCALE / (wsum + 1e-20)
    wa = vals[0] * inv
    wb = vals[1] * inv

    mask = sels[0] + sels[1]
    csum = mask
    sh = 1
    while sh < T:
        shifted = jnp.concatenate(
            [jnp.zeros((E, sh), jnp.float32), csum[:, :T - sh]], axis=1)
        csum = csum + shifted
        sh *= 2
    rank = csum - mask
    counts = csum[:, T - 1:T]
    padded = jnp.floor((counts + (BLK - 1)) * (1.0 / BLK)) * BLK
    ii = lax.broadcasted_iota(jnp.int32, (E, E), 0)
    jj = lax.broadcasted_iota(jnp.int32, (E, E), 1)
    lt = jnp.where(ii > jj, 1.0, 0.0)
    off = lax.dot_general(lt, padded, (((1,), (0,)), ((), ())),
                          preferred_element_type=jnp.float32)
    dest = off + rank
    da = jnp.sum(sels[0] * dest, axis=0, keepdims=True)
    db = jnp.sum(sels[1] * dest, axis=0, keepdims=True)
    da_i = da.astype(jnp.int32)
    db_i = db.astype(jnp.int32)
    da_r_ref[...] = da_i
    db_r_ref[...] = db_i
    da_c_ref[...] = jnp.transpose(da_i)
    db_c_ref[...] = jnp.transpose(db_i)
    wa_c_ref[...] = jnp.transpose(wa)
    wb_c_ref[...] = jnp.transpose(wb)

    nb = jnp.transpose(padded * (1.0 / BLK))
    off_t = jnp.transpose(off)
    meta_ref[...] = jnp.concatenate([nb, off_t], axis=1).astype(jnp.int32)


def _mlp(xbf, w1, w3, w2):
    h1 = lax.dot_general(xbf, w1.astype(jnp.bfloat16), (((1,), (1,)), ((), ())),
                         preferred_element_type=jnp.float32)
    h3 = lax.dot_general(xbf, w3.astype(jnp.bfloat16), (((1,), (1,)), ((), ())),
                         preferred_element_type=jnp.float32)
    act = (h1 * jax.nn.sigmoid(h1) * h3).astype(jnp.bfloat16)
    return lax.dot_general(act, w2.astype(jnp.bfloat16), (((1,), (1,)), ((), ())),
                           preferred_element_type=jnp.float32)


def _moe_kernel(meta_ref, xbf_ref, w1_ref, w3_ref, w2_ref, ws1_ref, ws3_ref,
                ws2_ref, da_r_ref, db_r_ref, da_c_ref, db_c_ref, wa_c_ref,
                wb_c_ref, y_ref):
    s = pl.program_id(0)

    @pl.when(s == 0)
    def _():
        y_ref[...] = _mlp(xbf_ref[...], ws1_ref[...], ws3_ref[...],
                          ws2_ref[...])

    @pl.when(s > 0)
    def _():
        e = s - 1
        nb = meta_ref[e]
        off = meta_ref[E + e]

        def body(i, carry):
            base = off + i * BLK
            rio = lax.broadcasted_iota(jnp.int32, (BLK, T), 0) + base
            pa = rio == da_r_ref[...]
            pb = rio == db_r_ref[...]
            p = jnp.where(pa, 1.0, 0.0) + jnp.where(pb, 1.0, 0.0)
            xs = lax.dot_general(p.astype(jnp.bfloat16), xbf_ref[...],
                                 (((1,), (0,)), ((), ())),
                                 preferred_element_type=jnp.float32)
            out = _mlp(xs.astype(jnp.bfloat16), w1_ref[0], w3_ref[0],
                       w2_ref[0])
            cval = (lax.dot_general(jnp.where(pa, 1.0, 0.0), wa_c_ref[...],
                                    (((1,), (0,)), ((), ())),
                                    preferred_element_type=jnp.float32)
                    + lax.dot_general(jnp.where(pb, 1.0, 0.0), wb_c_ref[...],
                                      (((1,), (0,)), ((), ())),
                                      preferred_element_type=jnp.float32))
            out_s = out * cval
            cio = lax.broadcasted_iota(jnp.int32, (T, BLK), 1) + base
            qa = cio == da_c_ref[...]
            qb = cio == db_c_ref[...]
            q = jnp.where(qa, 1.0, 0.0) + jnp.where(qb, 1.0, 0.0)
            y_ref[...] = y_ref[...] + lax.dot_general(
                q, out_s, (((1,), (0,)), ((), ())),
                preferred_element_type=jnp.float32)
            return carry

        lax.fori_loop(0, nb, body, 0)


@jax.jit
def kernel(x, gate_w, W1, W2, W3, Ws1, Ws2, Ws3):
    xbf, da_r, db_r, da_c, db_c, wa_c, wb_c, meta = pl.pallas_call(
        _gate_kernel,
        out_shape=(
            jax.ShapeDtypeStruct((T, D), jnp.bfloat16),
            jax.ShapeDtypeStruct((1, T), jnp.int32),
            jax.ShapeDtypeStruct((1, T), jnp.int32),
            jax.ShapeDtypeStruct((T, 1), jnp.int32),
            jax.ShapeDtypeStruct((T, 1), jnp.int32),
            jax.ShapeDtypeStruct((T, 1), jnp.float32),
            jax.ShapeDtypeStruct((T, 1), jnp.float32),
            jax.ShapeDtypeStruct((1, 2 * E), jnp.int32),
        ),
    )(x, gate_w)

    grid_spec = pltpu.PrefetchScalarGridSpec(
        num_scalar_prefetch=1,
        grid=(E + 1,),
        in_specs=[
            pl.BlockSpec((T, D), lambda s, m: (0, 0)),
            pl.BlockSpec((1, F, D), lambda s, m: (jnp.maximum(s - 1, 0), 0, 0)),
            pl.BlockSpec((1, F, D), lambda s, m: (jnp.maximum(s - 1, 0), 0, 0)),
            pl.BlockSpec((1, D, F), lambda s, m: (jnp.maximum(s - 1, 0), 0, 0)),
            pl.BlockSpec((F, D), lambda s, m: (0, 0)),
            pl.BlockSpec((F, D), lambda s, m: (0, 0)),
            pl.BlockSpec((D, F), lambda s, m: (0, 0)),
            pl.BlockSpec((1, T), lambda s, m: (0, 0)),
            pl.BlockSpec((1, T), lambda s, m: (0, 0)),
            pl.BlockSpec((T, 1), lambda s, m: (0, 0)),
            pl.BlockSpec((T, 1), lambda s, m: (0, 0)),
            pl.BlockSpec((T, 1), lambda s, m: (0, 0)),
            pl.BlockSpec((T, 1), lambda s, m: (0, 0)),
        ],
        out_specs=pl.BlockSpec((T, D), lambda s, m: (0, 0)),
    )
    y = pl.pallas_call(
        _moe_kernel,
        grid_spec=grid_spec,
        out_shape=jax.ShapeDtypeStruct((T, D), jnp.float32),
        compiler_params=pltpu.CompilerParams(
            dimension_semantics=("arbitrary",)),
    )(meta.reshape(2 * E), xbf, W1, W3, W2, Ws1, Ws3, Ws2,
      da_r, db_r, da_c, db_c, wa_c, wb_c)
    return y


# speedup vs baseline: 1.1419x; 1.0960x over previous
import jax
import jax.numpy as jnp
from jax import lax
from jax.experimental import pallas as pl
from jax.experimental.pallas import tpu as pltpu

E = 16
TOPK = 2
G = 8
KG = 4
ROUTE_SCALE = 2.5
D = 1024
F = 512
T = 1024

BLK = 128
NBLK = 32
_NEG = -1e30


def _gate_kernel(x_ref, gw_ref, xbf_ref, da_r_ref, db_r_ref, da_c_ref,
                 db_c_ref, wa_c_ref, wb_c_ref, meta_ref):
    x = x_ref[...]
    xbf_ref[...] = x.astype(jnp.bfloat16)
    scores = jax.nn.sigmoid(
        lax.dot_general(gw_ref[...], x, (((1,), (1,)), ((), ())),
                        preferred_element_type=jnp.float32))
    grows = [jnp.maximum(scores[2 * g:2 * g + 1, :], scores[2 * g + 1:2 * g + 2, :])
             for g in range(G)]
    gs = jnp.concatenate(grows, axis=0)
    iota_g = lax.broadcasted_iota(jnp.int32, (G, T), 0)
    keep = jnp.zeros((G, T), jnp.float32)
    work = gs
    for _ in range(KG):
        m = jnp.max(work, axis=0, keepdims=True)
        first = jnp.min(jnp.where(work == m, iota_g, G), axis=0, keepdims=True)
        sel = iota_g == first
        keep = keep + jnp.where(sel, 1.0, 0.0)
        work = jnp.where(sel, _NEG, work)
    keep_e = jnp.concatenate([keep[g:g + 1, :] for g in range(G) for _ in (0, 1)],
                             axis=0)
    masked = jnp.where(keep_e > 0.5, scores, _NEG)
    iota_e = lax.broadcasted_iota(jnp.int32, (E, T), 0)
    sels = []
    vals = []
    wsum = jnp.zeros((1, T), jnp.float32)
    work = masked
    for _ in range(TOPK):
        m = jnp.max(work, axis=0, keepdims=True)
        first = jnp.min(jnp.where(work == m, iota_e, E), axis=0, keepdims=True)
        sel = (iota_e == first)
        sels.append(jnp.where(sel, 1.0, 0.0))
        vals.append(m)
        wsum = wsum + m
        work = jnp.where(sel, _NEG, work)
    inv = ROUTE_SCALE / (wsum + 1e-20)
    wa = vals[0] * inv
    wb = vals[1] * inv

    mask = sels[0] + sels[1]
    csum = mask
    sh = 1
    while sh < T:
        shifted = jnp.concatenate(
            [jnp.zeros((E, sh), jnp.float32), csum[:, :T - sh]], axis=1)
        csum = csum + shifted
        sh *= 2
    rank = csum - mask
    counts = csum[:, T - 1:T]
    padded = jnp.floor((counts + (BLK - 1)) * (1.0 / BLK)) * BLK
    ii = lax.broadcasted_iota(jnp.int32, (E, E), 0)
    jj = lax.broadcasted_iota(jnp.int32, (E, E), 1)
    lt = jnp.where(ii > jj, 1.0, 0.0)
    off = lax.dot_general(lt, padded, (((1,), (0,)), ((), ())),
                          preferred_element_type=jnp.float32)
    dest = off + rank
    da = jnp.sum(sels[0] * dest, axis=0, keepdims=True)
    db = jnp.sum(sels[1] * dest, axis=0, keepdims=True)
    da_i = da.astype(jnp.int32)
    db_i = db.astype(jnp.int32)
    da_r_ref[...] = da_i
    db_r_ref[...] = db_i
    da_c_ref[...] = jnp.transpose(da_i)
    db_c_ref[...] = jnp.transpose(db_i)
    wa_c_ref[...] = jnp.transpose(wa)
    wb_c_ref[...] = jnp.transpose(wb)

    nb = jnp.transpose(padded * (1.0 / BLK))
    off_t = jnp.transpose(off)
    meta_ref[...] = jnp.concatenate([nb, off_t], axis=1).astype(jnp.int32)


def _mlp(xbf, w1, w3, w2):
    h1 = lax.dot_general(xbf, w1.astype(jnp.bfloat16), (((1,), (1,)), ((), ())),
                         preferred_element_type=jnp.float32)
    h3 = lax.dot_general(xbf, w3.astype(jnp.bfloat16), (((1,), (1,)), ((), ())),
                         preferred_element_type=jnp.float32)
    act = (h1 * jax.nn.sigmoid(h1) * h3).astype(jnp.bfloat16)
    return lax.dot_general(act, w2.astype(jnp.bfloat16), (((1,), (1,)), ((), ())),
                           preferred_element_type=jnp.float32)


def _moe_kernel(meta_ref, xbf_ref, w1_ref, w3_ref, w2_ref, ws1_ref, ws3_ref,
                ws2_ref, da_r_ref, db_r_ref, da_c_ref, db_c_ref, wa_c_ref,
                wb_c_ref, y_ref):
    s = pl.program_id(0)

    @pl.when(s == 0)
    def _():
        y_ref[...] = _mlp(xbf_ref[...], ws1_ref[...], ws3_ref[...],
                          ws2_ref[...])

    @pl.when(s > 0)
    def _():
        e = s - 1
        nb = meta_ref[e]
        off = meta_ref[E + e]

        def body(i, carry):
            base = off + i * BLK
            rio = lax.broadcasted_iota(jnp.int32, (BLK, T), 0) + base
            hit = jnp.logical_or(rio == da_r_ref[...], rio == db_r_ref[...])
            p = jnp.where(hit, 1.0, 0.0).astype(jnp.bfloat16)
            xs = lax.dot_general(p, xbf_ref[...], (((1,), (0,)), ((), ())),
                                 preferred_element_type=jnp.float32)
            out = _mlp(xs.astype(jnp.bfloat16), w1_ref[0], w3_ref[0],
                       w2_ref[0])
            cio = lax.broadcasted_iota(jnp.int32, (T, BLK), 1) + base
            q = (jnp.where(cio == da_c_ref[...], wa_c_ref[...], 0.0)
                 + jnp.where(cio == db_c_ref[...], wb_c_ref[...], 0.0))
            y_ref[...] = y_ref[...] + lax.dot_general(
                q, out, (((1,), (0,)), ((), ())),
                preferred_element_type=jnp.float32)
            return carry

        lax.fori_loop(0, nb, body, 0)


@jax.jit
def kernel(x, gate_w, W1, W2, W3, Ws1, Ws2, Ws3):
    xbf, da_r, db_r, da_c, db_c, wa_c, wb_c, meta = pl.pallas_call(
        _gate_kernel,
        out_shape=(
            jax.ShapeDtypeStruct((T, D), jnp.bfloat16),
            jax.ShapeDtypeStruct((1, T), jnp.int32),
            jax.ShapeDtypeStruct((1, T), jnp.int32),
            jax.ShapeDtypeStruct((T, 1), jnp.int32),
            jax.ShapeDtypeStruct((T, 1), jnp.int32),
            jax.ShapeDtypeStruct((T, 1), jnp.float32),
            jax.ShapeDtypeStruct((T, 1), jnp.float32),
            jax.ShapeDtypeStruct((1, 2 * E), jnp.int32),
        ),
    )(x, gate_w)

    grid_spec = pltpu.PrefetchScalarGridSpec(
        num_scalar_prefetch=1,
        grid=(E + 1,),
        in_specs=[
            pl.BlockSpec((T, D), lambda s, m: (0, 0)),
            pl.BlockSpec((1, F, D), lambda s, m: (jnp.maximum(s - 1, 0), 0, 0)),
            pl.BlockSpec((1, F, D), lambda s, m: (jnp.maximum(s - 1, 0), 0, 0)),
            pl.BlockSpec((1, D, F), lambda s, m: (jnp.maximum(s - 1, 0), 0, 0)),
            pl.BlockSpec((F, D), lambda s, m: (0, 0)),
            pl.BlockSpec((F, D), lambda s, m: (0, 0)),
            pl.BlockSpec((D, F), lambda s, m: (0, 0)),
            pl.BlockSpec((1, T), lambda s, m: (0, 0)),
            pl.BlockSpec((1, T), lambda s, m: (0, 0)),
            pl.BlockSpec((T, 1), lambda s, m: (0, 0)),
            pl.BlockSpec((T, 1), lambda s, m: (0, 0)),
            pl.BlockSpec((T, 1), lambda s, m: (0, 0)),
            pl.BlockSpec((T, 1), lambda s, m: (0, 0)),
        ],
        out_specs=pl.BlockSpec((T, D), lambda s, m: (0, 0)),
    )
    y = pl.pallas_call(
        _moe_kernel,
        grid_spec=grid_spec,
        out_shape=jax.ShapeDtypeStruct((T, D), jnp.float32),
        compiler_params=pltpu.CompilerParams(
            dimension_semantics=("arbitrary",)),
    )(meta.reshape(2 * E), xbf, W1, W3, W2, Ws1, Ws3, Ws2,
      da_r, db_r, da_c, db_c, wa_c, wb_c)
    return y


# speedup vs baseline: 1.4458x; 1.2661x over previous
import jax
import jax.numpy as jnp
from jax import lax
from jax.experimental import pallas as pl
from jax.experimental.pallas import tpu as pltpu

E = 16
TOPK = 2
G = 8
KG = 4
ROUTE_SCALE = 2.5
D = 1024
F = 512
T = 1024

BLK = 256
NBLK = 24
_NEG = -1e30


def _gate_kernel(x_ref, gw_ref, xbf_ref, da_r_ref, db_r_ref, da_c_ref,
                 db_c_ref, wa_c_ref, wb_c_ref, meta_ref):
    x = x_ref[...]
    xbf_ref[...] = x.astype(jnp.bfloat16)
    scores = jax.nn.sigmoid(
        lax.dot_general(gw_ref[...], x, (((1,), (1,)), ((), ())),
                        preferred_element_type=jnp.float32))
    grows = [jnp.maximum(scores[2 * g:2 * g + 1, :], scores[2 * g + 1:2 * g + 2, :])
             for g in range(G)]
    gs = jnp.concatenate(grows, axis=0)
    iota_g = lax.broadcasted_iota(jnp.int32, (G, T), 0)
    keep = jnp.zeros((G, T), jnp.float32)
    work = gs
    for _ in range(KG):
        m = jnp.max(work, axis=0, keepdims=True)
        first = jnp.min(jnp.where(work == m, iota_g, G), axis=0, keepdims=True)
        sel = iota_g == first
        keep = keep + jnp.where(sel, 1.0, 0.0)
        work = jnp.where(sel, _NEG, work)
    keep_e = jnp.concatenate([keep[g:g + 1, :] for g in range(G) for _ in (0, 1)],
                             axis=0)
    masked = jnp.where(keep_e > 0.5, scores, _NEG)
    iota_e = lax.broadcasted_iota(jnp.int32, (E, T), 0)
    sels = []
    vals = []
    wsum = jnp.zeros((1, T), jnp.float32)
    work = masked
    for _ in range(TOPK):
        m = jnp.max(work, axis=0, keepdims=True)
        first = jnp.min(jnp.where(work == m, iota_e, E), axis=0, keepdims=True)
        sel = (iota_e == first)
        sels.append(jnp.where(sel, 1.0, 0.0))
        vals.append(m)
        wsum = wsum + m
        work = jnp.where(sel, _NEG, work)
    inv = ROUTE_SCALE / (wsum + 1e-20)
    wa = vals[0] * inv
    wb = vals[1] * inv

    mask = sels[0] + sels[1]
    csum = mask
    sh = 1
    while sh < T:
        shifted = jnp.concatenate(
            [jnp.zeros((E, sh), jnp.float32), csum[:, :T - sh]], axis=1)
        csum = csum + shifted
        sh *= 2
    rank = csum - mask
    counts = csum[:, T - 1:T]
    padded = jnp.floor((counts + (BLK - 1)) * (1.0 / BLK)) * BLK
    ii = lax.broadcasted_iota(jnp.int32, (E, E), 0)
    jj = lax.broadcasted_iota(jnp.int32, (E, E), 1)
    lt = jnp.where(ii > jj, 1.0, 0.0)
    off = lax.dot_general(lt, padded, (((1,), (0,)), ((), ())),
                          preferred_element_type=jnp.float32)
    dest = off + rank
    da = jnp.sum(sels[0] * dest, axis=0, keepdims=True)
    db = jnp.sum(sels[1] * dest, axis=0, keepdims=True)
    da_i = da.astype(jnp.int32)
    db_i = db.astype(jnp.int32)
    da_r_ref[...] = da_i
    db_r_ref[...] = db_i
    da_c_ref[...] = jnp.transpose(da_i)
    db_c_ref[...] = jnp.transpose(db_i)
    wa_c_ref[...] = jnp.transpose(wa)
    wb_c_ref[...] = jnp.transpose(wb)

    nb = jnp.transpose(padded * (1.0 / BLK))
    off_t = jnp.transpose(off)
    meta_ref[...] = jnp.concatenate([nb, off_t], axis=1).astype(jnp.int32)


def _mlp(xbf, w1, w3, w2):
    h1 = lax.dot_general(xbf, w1.astype(jnp.bfloat16), (((1,), (1,)), ((), ())),
                         preferred_element_type=jnp.float32)
    h3 = lax.dot_general(xbf, w3.astype(jnp.bfloat16), (((1,), (1,)), ((), ())),
                         preferred_element_type=jnp.float32)
    act = (h1 * jax.nn.sigmoid(h1) * h3).astype(jnp.bfloat16)
    return lax.dot_general(act, w2.astype(jnp.bfloat16), (((1,), (1,)), ((), ())),
                           preferred_element_type=jnp.float32)


def _moe_kernel(meta_ref, xbf_ref, w1_ref, w3_ref, w2_ref, ws1_ref, ws3_ref,
                ws2_ref, da_r_ref, db_r_ref, da_c_ref, db_c_ref, wa_c_ref,
                wb_c_ref, y_ref):
    s = pl.program_id(0)

    @pl.when(s == 0)
    def _():
        y_ref[...] = _mlp(xbf_ref[...], ws1_ref[...], ws3_ref[...],
                          ws2_ref[...])

    @pl.when(s > 0)
    def _():
        e = s - 1
        nb = meta_ref[e]
        off = meta_ref[E + e]

        def body(i, carry):
            base = off + i * BLK
            rio = lax.broadcasted_iota(jnp.int32, (BLK, T), 0) + base
            hit = jnp.logical_or(rio == da_r_ref[...], rio == db_r_ref[...])
            p = jnp.where(hit, 1.0, 0.0).astype(jnp.bfloat16)
            xs = lax.dot_general(p, xbf_ref[...], (((1,), (0,)), ((), ())),
                                 preferred_element_type=jnp.float32)
            out = _mlp(xs.astype(jnp.bfloat16), w1_ref[0], w3_ref[0],
                       w2_ref[0])
            cio = lax.broadcasted_iota(jnp.int32, (T, BLK), 1) + base
            q = (jnp.where(cio == da_c_ref[...], wa_c_ref[...], 0.0)
                 + jnp.where(cio == db_c_ref[...], wb_c_ref[...], 0.0))
            y_ref[...] = y_ref[...] + lax.dot_general(
                q, out, (((1,), (0,)), ((), ())),
                preferred_element_type=jnp.float32)
            return carry

        lax.fori_loop(0, nb, body, 0)


@jax.jit
def kernel(x, gate_w, W1, W2, W3, Ws1, Ws2, Ws3):
    xbf, da_r, db_r, da_c, db_c, wa_c, wb_c, meta = pl.pallas_call(
        _gate_kernel,
        out_shape=(
            jax.ShapeDtypeStruct((T, D), jnp.bfloat16),
            jax.ShapeDtypeStruct((1, T), jnp.int32),
            jax.ShapeDtypeStruct((1, T), jnp.int32),
            jax.ShapeDtypeStruct((T, 1), jnp.int32),
            jax.ShapeDtypeStruct((T, 1), jnp.int32),
            jax.ShapeDtypeStruct((T, 1), jnp.float32),
            jax.ShapeDtypeStruct((T, 1), jnp.float32),
            jax.ShapeDtypeStruct((1, 2 * E), jnp.int32),
        ),
    )(x, gate_w)

    grid_spec = pltpu.PrefetchScalarGridSpec(
        num_scalar_prefetch=1,
        grid=(E + 1,),
        in_specs=[
            pl.BlockSpec((T, D), lambda s, m: (0, 0)),
            pl.BlockSpec((1, F, D), lambda s, m: (jnp.maximum(s - 1, 0), 0, 0)),
            pl.BlockSpec((1, F, D), lambda s, m: (jnp.maximum(s - 1, 0), 0, 0)),
            pl.BlockSpec((1, D, F), lambda s, m: (jnp.maximum(s - 1, 0), 0, 0)),
            pl.BlockSpec((F, D), lambda s, m: (0, 0)),
            pl.BlockSpec((F, D), lambda s, m: (0, 0)),
            pl.BlockSpec((D, F), lambda s, m: (0, 0)),
            pl.BlockSpec((1, T), lambda s, m: (0, 0)),
            pl.BlockSpec((1, T), lambda s, m: (0, 0)),
            pl.BlockSpec((T, 1), lambda s, m: (0, 0)),
            pl.BlockSpec((T, 1), lambda s, m: (0, 0)),
            pl.BlockSpec((T, 1), lambda s, m: (0, 0)),
            pl.BlockSpec((T, 1), lambda s, m: (0, 0)),
        ],
        out_specs=pl.BlockSpec((T, D), lambda s, m: (0, 0)),
    )
    y = pl.pallas_call(
        _moe_kernel,
        grid_spec=grid_spec,
        out_shape=jax.ShapeDtypeStruct((T, D), jnp.float32),
        compiler_params=pltpu.CompilerParams(
            dimension_semantics=("arbitrary",)),
    )(meta.reshape(2 * E), xbf, W1, W3, W2, Ws1, Ws3, Ws2,
      da_r, db_r, da_c, db_c, wa_c, wb_c)
    return y
